# baseline (speedup 1.0000x reference)
import functools
import math

import jax
import jax.numpy as jnp
from jax import lax
from jax.experimental import pallas as pl
from jax.experimental.pallas import tpu as pltpu
from jax.experimental.pallas import tpu_sc as plsc

N = 50000
E = 800000
L = 3
H = 128
F = 128
G = 50
NG = 500
NTYPES = 100
CUTOFF = 5.0
NFC = 2
NCLS = 1

LN2 = math.log(2.0)

NODE_BLK = 5000
EDGE_BLK = 10000

TAB_K = 16384
TAB_STEP = CUTOFF / TAB_K
TAB_ROWS = 16512
TAB_BUILD = TAB_ROWS + 8
TAB_REPS = 8


def _ssp(v):
    return jax.nn.softplus(v) - LN2


def _embed_body(x_ref, emb_ref, w1_ref, h_ref, m_ref):
    xv = x_ref[...]
    ids = lax.broadcasted_iota(jnp.int32, (NODE_BLK, 128), 1).astype(jnp.float32)
    oh = jnp.where(ids == xv, 1.0, 0.0)
    h = jnp.dot(oh, emb_ref[...], preferred_element_type=jnp.float32)
    h_ref[...] = h
    m_ref[...] = jnp.dot(h, w1_ref[...], preferred_element_type=jnp.float32)


def _tc_embed(x_f, emb_p, w1_0):
    return pl.pallas_call(
        _embed_body,
        grid=(N // NODE_BLK,),
        in_specs=[
            pl.BlockSpec((NODE_BLK, 1), lambda i: (i, 0)),
            pl.BlockSpec((128, 128), lambda i: (0, 0)),
            pl.BlockSpec((128, 128), lambda i: (0, 0)),
        ],
        out_specs=[
            pl.BlockSpec((NODE_BLK, 128), lambda i: (i, 0)),
            pl.BlockSpec((NODE_BLK, 128), lambda i: (i, 0)),
        ],
        out_shape=[
            jax.ShapeDtypeStruct((N, 128), jnp.float32),
            jax.ShapeDtypeStruct((N, 128), jnp.float32),
        ],
    )(x_f, emb_p, w1_0)


def _table_body(d_ref, wf1_ref, bf1_ref, wf2_ref, bf2_ref, out_ref):
    d = d_ref[...]
    step = CUTOFF / (G - 1)
    offs = lax.broadcasted_iota(jnp.int32, (1, 128), 1).astype(jnp.float32) * step
    coeff = -0.5 / (step * step)
    rbf = jnp.exp(coeff * (d - offs) ** 2)
    t = _ssp(jnp.dot(rbf, wf1_ref[...], preferred_element_type=jnp.float32)
             + bf1_ref[...][0:1])
    w = (jnp.dot(t, wf2_ref[...], preferred_element_type=jnp.float32)
         + bf2_ref[...][0:1])
    c = 0.5 * (jnp.cos(d * (math.pi / CUTOFF)) + 1.0)
    c = jnp.where(d < CUTOFF, c, 0.0)
    out_ref[...] = w * c


def _tc_table(dgrid, wf1_l, bf1_l, wf2_l, bf2_l):
    return pl.pallas_call(
        _table_body,
        grid=(1,),
        in_specs=[
            pl.BlockSpec((TAB_BUILD, 128), lambda i: (0, 0)),
            pl.BlockSpec((128, 128), lambda i: (0, 0)),
            pl.BlockSpec((8, 128), lambda i: (0, 0)),
            pl.BlockSpec((128, 128), lambda i: (0, 0)),
            pl.BlockSpec((8, 128), lambda i: (0, 0)),
        ],
        out_specs=pl.BlockSpec((TAB_BUILD, 128), lambda i: (0, 0)),
        out_shape=jax.ShapeDtypeStruct((TAB_BUILD, 128), jnp.float32),
    )(dgrid, wf1_l, bf1_l, wf2_l, bf2_l)


UROWS = E // 8


def _edge_u_body(ps_ref, pd_ref, seg_ref, out_ref):
    diff = ps_ref[...] - pd_ref[...]
    d2 = jnp.dot(diff * diff, seg_ref[...],
                 preferred_element_type=jnp.float32) + 1e-12
    u = jnp.minimum(jnp.sqrt(d2) * (1.0 / TAB_STEP), float(TAB_K))
    out_ref[...] = u[:, 0:8]


def _tc_edge_u(ps8, pd8, seg):
    blk = 10000
    return pl.pallas_call(
        _edge_u_body,
        grid=(UROWS // blk,),
        in_specs=[
            pl.BlockSpec((blk, 128), lambda i: (i, 0)),
            pl.BlockSpec((blk, 128), lambda i: (i, 0)),
            pl.BlockSpec((128, 128), lambda i: (0, 0)),
        ],
        out_specs=pl.BlockSpec((blk, 8), lambda i: (i, 0)),
        out_shape=jax.ShapeDtypeStruct((UROWS, 8), jnp.float32),
    )(ps8, pd8, seg)


def _node_body_mid(agg_ref, h_ref, w2_ref, b2_ref, w3_ref, b3_ref, w1n_ref,
                   hn_ref, mn_ref):
    v = _ssp(jnp.dot(agg_ref[...], w2_ref[...],
                     preferred_element_type=jnp.float32) + b2_ref[...][0:1])
    hn = h_ref[...] + jnp.dot(v, w3_ref[...],
                              preferred_element_type=jnp.float32) + b3_ref[...][0:1]
    hn_ref[...] = hn
    mn_ref[...] = jnp.dot(hn, w1n_ref[...], preferred_element_type=jnp.float32)


def _node_body_last(agg_ref, h_ref, w2_ref, b2_ref, w3_ref, b3_ref, hn_ref):
    v = _ssp(jnp.dot(agg_ref[...], w2_ref[...],
                     preferred_element_type=jnp.float32) + b2_ref[...][0:1])
    hn_ref[...] = h_ref[...] + jnp.dot(v, w3_ref[...],
                                       preferred_element_type=jnp.float32) + b3_ref[...][0:1]


def _tc_node_update(agg, h, w2_l, b2_l, w3_l, b3_l, w1_next):
    full = lambda i: (0, 0)
    blk = lambda i: (i, 0)
    if w1_next is not None:
        return pl.pallas_call(
            _node_body_mid,
            grid=(N // NODE_BLK,),
            in_specs=[
                pl.BlockSpec((NODE_BLK, 128), blk),
                pl.BlockSpec((NODE_BLK, 128), blk),
                pl.BlockSpec((128, 128), full),
                pl.BlockSpec((8, 128), full),
                pl.BlockSpec((128, 128), full),
                pl.BlockSpec((8, 128), full),
                pl.BlockSpec((128, 128), full),
            ],
            out_specs=[
                pl.BlockSpec((NODE_BLK, 128), blk),
                pl.BlockSpec((NODE_BLK, 128), blk),
            ],
            out_shape=[
                jax.ShapeDtypeStruct((N, 128), jnp.float32),
                jax.ShapeDtypeStruct((N, 128), jnp.float32),
            ],
        )(agg, h, w2_l, b2_l, w3_l, b3_l, w1_next)
    return pl.pallas_call(
        _node_body_last,
        grid=(N // NODE_BLK,),
        in_specs=[
            pl.BlockSpec((NODE_BLK, 128), blk),
            pl.BlockSpec((NODE_BLK, 128), blk),
            pl.BlockSpec((128, 128), full),
            pl.BlockSpec((8, 128), full),
            pl.BlockSpec((128, 128), full),
            pl.BlockSpec((8, 128), full),
        ],
        out_specs=pl.BlockSpec((NODE_BLK, 128), blk),
        out_shape=jax.ShapeDtypeStruct((N, 128), jnp.float32),
    )(agg, h, w2_l, b2_l, w3_l, b3_l)


def _readout_body(h_ref, b_ref, fw0_ref, fb0_ref, fw1_ref, fb1_ref,
                  ow_ref, ob_ref, out_ref, sums_ref, cnts_ref):
    i = pl.program_id(0)
    nblk = pl.num_programs(0)

    @pl.when(i == 0)
    def _():
        sums_ref[...] = jnp.zeros_like(sums_ref)
        cnts_ref[...] = jnp.zeros_like(cnts_ref)

    bv = b_ref[...]
    gids = lax.broadcasted_iota(jnp.int32, (NODE_BLK, 512), 1).astype(jnp.float32)
    oh = jnp.where(gids == bv, 1.0, 0.0)
    hv = h_ref[...]
    dn = (((0,), (0,)), ((), ()))
    sums_ref[...] += lax.dot_general(oh, hv, dn,
                                     preferred_element_type=jnp.float32)
    cnts_ref[...] += lax.dot_general(oh, jnp.ones_like(hv), dn,
                                     preferred_element_type=jnp.float32)

    @pl.when(i == nblk - 1)
    def _():
        g = sums_ref[...] / jnp.maximum(cnts_ref[...], 1.0)
        g = jax.nn.gelu(jnp.dot(g, fw0_ref[...],
                                preferred_element_type=jnp.float32)
                        + fb0_ref[...][0:1])
        g = jax.nn.gelu(jnp.dot(g, fw1_ref[...],
                                preferred_element_type=jnp.float32)
                        + fb1_ref[...][0:1])
        out_ref[...] = jnp.dot(g, ow_ref[...],
                               preferred_element_type=jnp.float32) + ob_ref[...][0:1]


def _tc_readout(h, batch_f, fw0, fb0, fw1, fb1, ow_p, ob_p):
    full = lambda i: (0, 0)
    return pl.pallas_call(
        _readout_body,
        grid=(N // NODE_BLK,),
        in_specs=[
            pl.BlockSpec((NODE_BLK, 128), lambda i: (i, 0)),
            pl.BlockSpec((NODE_BLK, 1), lambda i: (i, 0)),
            pl.BlockSpec((128, 128), full),
            pl.BlockSpec((8, 128), full),
            pl.BlockSpec((128, 128), full),
            pl.BlockSpec((8, 128), full),
            pl.BlockSpec((128, 128), full),
            pl.BlockSpec((8, 128), full),
        ],
        out_specs=pl.BlockSpec((512, 128), full),
        out_shape=jax.ShapeDtypeStruct((512, 128), jnp.float32),
        scratch_shapes=[
            pltpu.VMEM((512, 128), jnp.float32),
            pltpu.VMEM((512, 128), jnp.float32),
        ],
    )(h, batch_f, fw0, fb0, fw1, fb1, ow_p, ob_p)


def _rep8(b):
    return jnp.broadcast_to(b[None, :], (8, b.shape[0])).astype(jnp.float32)


_MESH = plsc.VectorSubcoreMesh(core_axis_name="c", subcore_axis_name="s")
TILES = 32
EPT = E // TILES
NQ = 4
QN = N // NQ
SP_ROWS = QN + 44
STRIPE = SP_ROWS // 16
SLOT = EPT + 128
CC_CHUNK = 1000
CC_VECS = 63
MSG_CHUNK = 48
SUPER = 8


def _sc_pos_gather(pos16, src, dst):
    @functools.partial(
        pl.kernel,
        out_type=[jax.ShapeDtypeStruct((E, 16), jnp.float32),
                  jax.ShapeDtypeStruct((E, 16), jnp.float32)],
        mesh=_MESH,
        scratch_types=[pltpu.VMEM((1000,), jnp.int32),
                       pltpu.VMEM((1000, 16), jnp.float32),
                       pltpu.SemaphoreType.DMA],
        compiler_params=pltpu.CompilerParams(use_tc_tiling_on_sc=False),
    )
    def k(pos_h, src_h, dst_h, ps_o, pd_o, idx_v, rows_v, sem):
        wid = lax.axis_index("c") * 16 + lax.axis_index("s")
        base = wid * EPT
        for idx_h, out_h in ((src_h, ps_o), (dst_h, pd_o)):
            def body(i, _, idx_h=idx_h, out_h=out_h):
                off = base + i * 1000
                pltpu.sync_copy(idx_h.at[pl.ds(off, 1000)], idx_v)
                cps = []
                for kk in range(7):
                    cps.append(pltpu.async_copy(
                        pos_h.at[idx_v.at[pl.ds(kk * 128, 128)]],
                        rows_v.at[pl.ds(kk * 128, 128)], sem))
                cps.append(pltpu.async_copy(
                    pos_h.at[idx_v.at[pl.ds(896, 104)]],
                    rows_v.at[pl.ds(896, 104)], sem))
                for cp in cps:
                    cp.wait()
                pltpu.sync_copy(rows_v, out_h.at[pl.ds(off, 1000)])
                return 0
            lax.fori_loop(0, EPT // 1000, body, 0)

    return k(pos16, src, dst)


def _sc_compact(src, dst, u):
    @functools.partial(
        pl.kernel,
        out_type=[jax.ShapeDtypeStruct((NQ * TILES * SLOT,), jnp.int32),
                  jax.ShapeDtypeStruct((NQ * TILES * SLOT,), jnp.int32),
                  jax.ShapeDtypeStruct((NQ * TILES * SLOT,), jnp.float32),
                  jax.ShapeDtypeStruct((NQ * TILES * 8 + 8,), jnp.int32)],
        mesh=_MESH,
        scratch_types=[pltpu.VMEM((1008,), jnp.int32),
                       pltpu.VMEM((1008,), jnp.int32),
                       pltpu.VMEM((1008,), jnp.float32),
                       pltpu.VMEM((SLOT + 16,), jnp.int32),
                       pltpu.VMEM((SLOT + 16,), jnp.int32),
                       pltpu.VMEM((SLOT + 16,), jnp.float32),
                       pltpu.VMEM((16,), jnp.int32)],
        compiler_params=pltpu.CompilerParams(use_tc_tiling_on_sc=False,
                                             needs_layout_passes=False),
    )
    def k(src_h, dst_h, u_h, csrc_o, cdst_o, cu_o, cnt_o,
          s_in, d_in, u_in, bsrc, bdst, bu, cnt_v):
        wid = lax.axis_index("c") * 16 + lax.axis_index("s")
        base = wid * EPT
        lane = lax.broadcasted_iota(jnp.int32, (16,), 0)
        for q in range(NQ):
            lo = q * QN
            hi = lo + QN

            def chunk_body(c, off, lo=lo, hi=hi):
                pltpu.sync_copy(src_h.at[pl.ds(base + c * CC_CHUNK, CC_CHUNK)],
                                s_in.at[pl.ds(0, CC_CHUNK)])
                pltpu.sync_copy(dst_h.at[pl.ds(base + c * CC_CHUNK, CC_CHUNK)],
                                d_in.at[pl.ds(0, CC_CHUNK)])
                pltpu.sync_copy(u_h.at[pl.ds(base + c * CC_CHUNK, CC_CHUNK)],
                                u_in.at[pl.ds(0, CC_CHUNK)])

                def vec_body(kk, off2):
                    sv = s_in[pl.ds(kk * 16, 16)]
                    dv = d_in[pl.ds(kk * 16, 16)]
                    uv = u_in[pl.ds(kk * 16, 16)]
                    valid = lane < (CC_CHUNK - kk * 16)
                    msk = valid & (dv >= lo) & (dv < hi)
                    mi = msk.astype(jnp.int32)
                    ics = plsc.cumsum(mi)
                    idx = jnp.where(msk, off2 + ics - mi, SLOT + lane)
                    plsc.store_scatter(bsrc, [idx], sv)
                    plsc.store_scatter(bdst, [idx], dv - lo)
                    plsc.store_scatter(bu, [idx], uv)
                    return off2 + ics[15]

                return lax.fori_loop(0, CC_VECS, vec_body, off)

            off = lax.fori_loop(0, EPT // CC_CHUNK, chunk_body, 0)
            dump_d = QN + (lane & 7)
            zero16 = jnp.zeros((16,), jnp.int32)
            ktop16 = jnp.full((16,), float(TAB_K), jnp.float32)
            for j in range(3):
                bsrc[pl.ds(off + j * 16, 16)] = zero16
                bdst[pl.ds(off + j * 16, 16)] = dump_d
                bu[pl.ds(off + j * 16, 16)] = ktop16
            off_pad = ((off + MSG_CHUNK - 1) // MSG_CHUNK) * MSG_CHUNK
            cnt_v[...] = jnp.full((16,), off_pad, jnp.int32)
            pltpu.sync_copy(cnt_v.at[pl.ds(0, 8)],
                            cnt_o.at[pl.ds((q * TILES) * 8 + wid * 8, 8)])
            qbase = q * TILES * SLOT
            pltpu.sync_copy(bsrc.at[pl.ds(0, SLOT)],
                            csrc_o.at[pl.ds(qbase + wid * SLOT, SLOT)])
            pltpu.sync_copy(bdst.at[pl.ds(0, SLOT)],
                            cdst_o.at[pl.ds(qbase + wid * SLOT, SLOT)])
            pltpu.sync_copy(bu.at[pl.ds(0, SLOT)],
                            cu_o.at[pl.ds(qbase + wid * SLOT, SLOT)])

    return k(src, dst, u)


def _sc_message(m, tpair, csrc, cdst, cu, counts, zeros_buf):
    @functools.partial(
        pl.kernel,
        out_type=jax.ShapeDtypeStruct((NQ * SP_ROWS, 128), jnp.float32),
        mesh=_MESH,
        scratch_types=[pltpu.VMEM_SHARED((SP_ROWS, 128), jnp.float32),
                       pltpu.VMEM((NQ * TILES * 8 + 8,), jnp.int32),
                       pltpu.VMEM((SUPER * MSG_CHUNK,), jnp.int32),
                       pltpu.VMEM((SUPER * MSG_CHUNK,), jnp.int32),
                       pltpu.VMEM((SUPER * MSG_CHUNK,), jnp.float32),
                       pltpu.VMEM((SUPER * MSG_CHUNK,), jnp.int32),
                       pltpu.VMEM((MSG_CHUNK, 128), jnp.float32),
                       pltpu.VMEM((MSG_CHUNK, 128), jnp.float32),
                       pltpu.VMEM((MSG_CHUNK, 128), jnp.float32),
                       pltpu.VMEM((MSG_CHUNK, 128), jnp.float32),
                       pltpu.SemaphoreType.DMA,
                       pltpu.SemaphoreType.DMA],
        compiler_params=pltpu.CompilerParams(needs_layout_passes=False),
    )
    def k(m_h, tp_h, csrc_h, cdst_h, cu_h, cnt_h, zeros_h, agg_h,
          sharedA, cnt_v, src_v, dst_v, u_v, k_v, mrow0, prow0, mrow1, prow1,
          semA, semB):
        cid = lax.axis_index("c")
        sid = lax.axis_index("s")
        lane = lax.broadcasted_iota(jnp.int32, (16,), 0)
        rep_off = (lane & (TAB_REPS - 1)) * TAB_ROWS
        pltpu.sync_copy(cnt_h, cnt_v)
        mrows = (mrow0, mrow1)
        prows = (prow0, prow1)
        sems = (semA, semB)

        def mul_scatter(slot, sub):
            mrow, prow = mrows[slot], prows[slot]

            def mul(j, _):
                for cc in range(8):
                    sl = pl.ds(cc * 16, 16)
                    mrow[j, sl] = mrow[j, sl] * prow[j, sl]
                return 0

            lax.fori_loop(0, MSG_CHUNK, mul, 0)
            pltpu.sync_copy(mrow,
                            sharedA.at[dst_v.at[pl.ds(sub * MSG_CHUNK,
                                                      MSG_CHUNK)]],
                            add=True)

        def issue(sub, slot):
            s = pl.ds(sub * MSG_CHUNK, MSG_CHUNK)
            c1 = pltpu.async_copy(m_h.at[src_v.at[s]], mrows[slot],
                                  sems[slot])
            c2 = pltpu.async_copy(tp_h.at[k_v.at[s]], prows[slot],
                                  sems[slot])
            return (c1, c2)

        for qj in range(NQ // 2):
            q = cid * (NQ // 2) + qj
            pltpu.sync_copy(zeros_h,
                            sharedA.at[pl.ds(sid * STRIPE, STRIPE)])
            plsc.subcore_barrier()
            for tj in range(2):
                t = tj * 16 + sid
                nq = cnt_v[pl.ds((q * TILES + t) * 8, 16)][0]
                trips = nq // MSG_CHUNK
                nss = trips // SUPER

                def load_idx(b, count):
                    pltpu.sync_copy(csrc_h.at[pl.ds(b, count)],
                                    src_v.at[pl.ds(0, count)])
                    pltpu.sync_copy(cdst_h.at[pl.ds(b, count)],
                                    dst_v.at[pl.ds(0, count)])
                    pltpu.sync_copy(cu_h.at[pl.ds(b, count)],
                                    u_v.at[pl.ds(0, count)])
                    for g in range(count // 16):
                        u16 = u_v[pl.ds(g * 16, 16)]
                        k16 = (u16 + 0.5).astype(jnp.int32) + rep_off
                        k_v[pl.ds(g * 16, 16)] = k16

                def ss_body(si, _, t=t, q=q):
                    b = q * TILES * SLOT + t * SLOT + si * (SUPER * MSG_CHUNK)
                    load_idx(b, SUPER * MSG_CHUNK)
                    cps = {0: issue(0, 0)}
                    for sub in range(SUPER):
                        slot = sub & 1
                        if sub + 1 < SUPER:
                            cps[sub + 1] = issue(sub + 1, slot ^ 1)
                        cps[sub][0].wait()
                        cps[sub][1].wait()
                        mul_scatter(slot, sub)
                    return 0

                lax.fori_loop(0, nss, ss_body, 0)

                def tail_body(ci, _, t=t, q=q):
                    b = (q * TILES * SLOT + t * SLOT + ci * MSG_CHUNK)
                    load_idx(b, MSG_CHUNK)
                    c1, c2 = issue(0, 0)
                    c1.wait()
                    c2.wait()
                    mul_scatter(0, 0)
                    return 0

                lax.fori_loop(nss * SUPER, trips, tail_body, 0)
            plsc.subcore_barrier()
            row0 = q * SP_ROWS + sid * STRIPE
            pltpu.sync_copy(sharedA.at[pl.ds(sid * STRIPE, STRIPE)],
                            agg_h.at[pl.ds(row0, STRIPE)])
            plsc.subcore_barrier()

    return k(m, tpair, csrc, cdst, cu, counts, zeros_buf)


def kernel(pos, emb, Wf1, bf1, Wf2, bf2, W1, W2, b2, W3, b3, fcW, fcb,
           outW, outb, x, edge_index, batch):
    src = edge_index[0]
    dst = edge_index[1]
    x_f = x.astype(jnp.float32)
    batch_f = batch.astype(jnp.float32)[:, None]
    emb_p = jnp.pad(emb, ((0, 128 - NTYPES), (0, 0)))
    wf1_p = jnp.pad(Wf1, ((0, 0), (0, 128 - G), (0, 0)))
    ow_p = jnp.pad(outW, ((0, 0), (0, 128 - NCLS)))
    ob_p = _rep8(jnp.pad(outb, (0, 128 - NCLS)))

    h, m = _tc_embed(x_f, emb_p, W1[0])

    pos16 = jnp.pad(pos, ((0, 0), (0, 13)))
    pos_s, pos_d = _sc_pos_gather(pos16, src, dst)
    seg = ((jnp.arange(128)[:, None] // 16 == jnp.arange(128)[None, :])
           & (jnp.arange(128)[None, :] < 8)).astype(jnp.float32)
    ps8 = jnp.reshape(pos_s, (UROWS, 128))
    pd8 = jnp.reshape(pos_d, (UROWS, 128))
    u8 = _tc_edge_u(ps8, pd8, seg)
    u1 = jnp.reshape(u8, (E,))
    csrc, cdst, cu, counts = _sc_compact(src, dst, u1)
    zeros_buf = jnp.zeros((STRIPE, 128), jnp.float32)

    dgrid = jnp.broadcast_to(
        (jnp.arange(TAB_BUILD, dtype=jnp.float32) * TAB_STEP)[:, None],
        (TAB_BUILD, 128))

    for l in range(L):
        tab = _tc_table(dgrid, wf1_p[l], _rep8(bf1[l]), Wf2[l], _rep8(bf2[l]))
        trep = jnp.tile(tab[0:TAB_ROWS], (TAB_REPS, 1))
        agg_full = _sc_message(m, trep, csrc, cdst, cu, counts, zeros_buf)
        agg = jnp.concatenate(
            [agg_full[q * SP_ROWS:q * SP_ROWS + QN] for q in range(NQ)], axis=0)
        w1n = W1[l + 1] if l + 1 < L else None
        if w1n is not None:
            h, m = _tc_node_update(agg, h, W2[l], _rep8(b2[l]), W3[l],
                                   _rep8(b3[l]), w1n)
        else:
            h = _tc_node_update(agg, h, W2[l], _rep8(b2[l]), W3[l],
                                _rep8(b3[l]), None)

    out = _tc_readout(h, batch_f, fcW[0], _rep8(fcb[0]), fcW[1],
                      _rep8(fcb[1]), ow_p, ob_p)
    return out[:NG, :NCLS]


# speedup vs baseline: 1.0371x; 1.0371x over previous
import functools
import math

import jax
import jax.numpy as jnp
from jax import lax
from jax.experimental import pallas as pl
from jax.experimental.pallas import tpu as pltpu
from jax.experimental.pallas import tpu_sc as plsc

N = 50000
E = 800000
L = 3
H = 128
F = 128
G = 50
NG = 500
NTYPES = 100
CUTOFF = 5.0
NFC = 2
NCLS = 1

LN2 = math.log(2.0)

NODE_BLK = 5000
EDGE_BLK = 10000

TAB_K = 16384
TAB_STEP = CUTOFF / TAB_K
TAB_ROWS = 16512
TAB_BUILD = TAB_ROWS + 8
TAB_REPS = 8


def _ssp(v):
    return jax.nn.softplus(v) - LN2


def _embed_body(x_ref, emb_ref, w1_ref, h_ref, m_ref):
    xv = x_ref[...]
    ids = lax.broadcasted_iota(jnp.int32, (NODE_BLK, 128), 1).astype(jnp.float32)
    oh = jnp.where(ids == xv, 1.0, 0.0)
    h = jnp.dot(oh, emb_ref[...], preferred_element_type=jnp.float32)
    h_ref[...] = h
    m_ref[...] = jnp.dot(h, w1_ref[...], preferred_element_type=jnp.float32)


def _tc_embed(x_f, emb_p, w1_0):
    return pl.pallas_call(
        _embed_body,
        grid=(N // NODE_BLK,),
        in_specs=[
            pl.BlockSpec((NODE_BLK, 1), lambda i: (i, 0)),
            pl.BlockSpec((128, 128), lambda i: (0, 0)),
            pl.BlockSpec((128, 128), lambda i: (0, 0)),
        ],
        out_specs=[
            pl.BlockSpec((NODE_BLK, 128), lambda i: (i, 0)),
            pl.BlockSpec((NODE_BLK, 128), lambda i: (i, 0)),
        ],
        out_shape=[
            jax.ShapeDtypeStruct((N, 128), jnp.float32),
            jax.ShapeDtypeStruct((N, 128), jnp.float32),
        ],
    )(x_f, emb_p, w1_0)


def _table_body(d_ref, wf1_ref, bf1_ref, wf2_ref, bf2_ref, out_ref):
    d = d_ref[...]
    step = CUTOFF / (G - 1)
    offs = lax.broadcasted_iota(jnp.int32, (1, 128), 1).astype(jnp.float32) * step
    coeff = -0.5 / (step * step)
    rbf = jnp.exp(coeff * (d - offs) ** 2)
    t = _ssp(jnp.dot(rbf, wf1_ref[...], preferred_element_type=jnp.float32)
             + bf1_ref[...][0:1])
    w = (jnp.dot(t, wf2_ref[...], preferred_element_type=jnp.float32)
         + bf2_ref[...][0:1])
    c = 0.5 * (jnp.cos(d * (math.pi / CUTOFF)) + 1.0)
    c = jnp.where(d < CUTOFF, c, 0.0)
    out_ref[...] = w * c


def _tc_table(dgrid, wf1_l, bf1_l, wf2_l, bf2_l):
    return pl.pallas_call(
        _table_body,
        grid=(1,),
        in_specs=[
            pl.BlockSpec((TAB_BUILD, 128), lambda i: (0, 0)),
            pl.BlockSpec((128, 128), lambda i: (0, 0)),
            pl.BlockSpec((8, 128), lambda i: (0, 0)),
            pl.BlockSpec((128, 128), lambda i: (0, 0)),
            pl.BlockSpec((8, 128), lambda i: (0, 0)),
        ],
        out_specs=pl.BlockSpec((TAB_BUILD, 128), lambda i: (0, 0)),
        out_shape=jax.ShapeDtypeStruct((TAB_BUILD, 128), jnp.float32),
    )(dgrid, wf1_l, bf1_l, wf2_l, bf2_l)


UROWS = E // 8


def _edge_u_body(ps_ref, pd_ref, seg_ref, out_ref):
    diff = ps_ref[...] - pd_ref[...]
    d2 = jnp.dot(diff * diff, seg_ref[...],
                 preferred_element_type=jnp.float32) + 1e-12
    u = jnp.minimum(jnp.sqrt(d2) * (1.0 / TAB_STEP), float(TAB_K))
    out_ref[...] = u[:, 0:8]


def _tc_edge_u(ps8, pd8, seg):
    blk = 10000
    return pl.pallas_call(
        _edge_u_body,
        grid=(UROWS // blk,),
        in_specs=[
            pl.BlockSpec((blk, 128), lambda i: (i, 0)),
            pl.BlockSpec((blk, 128), lambda i: (i, 0)),
            pl.BlockSpec((128, 128), lambda i: (0, 0)),
        ],
        out_specs=pl.BlockSpec((blk, 8), lambda i: (i, 0)),
        out_shape=jax.ShapeDtypeStruct((UROWS, 8), jnp.float32),
    )(ps8, pd8, seg)


def _node_body_mid(agg_ref, h_ref, w2_ref, b2_ref, w3_ref, b3_ref, w1n_ref,
                   hn_ref, mn_ref):
    v = _ssp(jnp.dot(agg_ref[...], w2_ref[...],
                     preferred_element_type=jnp.float32) + b2_ref[...][0:1])
    hn = h_ref[...] + jnp.dot(v, w3_ref[...],
                              preferred_element_type=jnp.float32) + b3_ref[...][0:1]
    hn_ref[...] = hn
    mn_ref[...] = jnp.dot(hn, w1n_ref[...], preferred_element_type=jnp.float32)


def _node_body_last(agg_ref, h_ref, w2_ref, b2_ref, w3_ref, b3_ref, hn_ref):
    v = _ssp(jnp.dot(agg_ref[...], w2_ref[...],
                     preferred_element_type=jnp.float32) + b2_ref[...][0:1])
    hn_ref[...] = h_ref[...] + jnp.dot(v, w3_ref[...],
                                       preferred_element_type=jnp.float32) + b3_ref[...][0:1]


def _tc_node_update(agg, h, w2_l, b2_l, w3_l, b3_l, w1_next):
    full = lambda i: (0, 0)
    blk = lambda i: (i, 0)
    if w1_next is not None:
        return pl.pallas_call(
            _node_body_mid,
            grid=(N // NODE_BLK,),
            in_specs=[
                pl.BlockSpec((NODE_BLK, 128), blk),
                pl.BlockSpec((NODE_BLK, 128), blk),
                pl.BlockSpec((128, 128), full),
                pl.BlockSpec((8, 128), full),
                pl.BlockSpec((128, 128), full),
                pl.BlockSpec((8, 128), full),
                pl.BlockSpec((128, 128), full),
            ],
            out_specs=[
                pl.BlockSpec((NODE_BLK, 128), blk),
                pl.BlockSpec((NODE_BLK, 128), blk),
            ],
            out_shape=[
                jax.ShapeDtypeStruct((N, 128), jnp.float32),
                jax.ShapeDtypeStruct((N, 128), jnp.float32),
            ],
        )(agg, h, w2_l, b2_l, w3_l, b3_l, w1_next)
    return pl.pallas_call(
        _node_body_last,
        grid=(N // NODE_BLK,),
        in_specs=[
            pl.BlockSpec((NODE_BLK, 128), blk),
            pl.BlockSpec((NODE_BLK, 128), blk),
            pl.BlockSpec((128, 128), full),
            pl.BlockSpec((8, 128), full),
            pl.BlockSpec((128, 128), full),
            pl.BlockSpec((8, 128), full),
        ],
        out_specs=pl.BlockSpec((NODE_BLK, 128), blk),
        out_shape=jax.ShapeDtypeStruct((N, 128), jnp.float32),
    )(agg, h, w2_l, b2_l, w3_l, b3_l)


def _readout_body(h_ref, b_ref, fw0_ref, fb0_ref, fw1_ref, fb1_ref,
                  ow_ref, ob_ref, out_ref, sums_ref, cnts_ref):
    i = pl.program_id(0)
    nblk = pl.num_programs(0)

    @pl.when(i == 0)
    def _():
        sums_ref[...] = jnp.zeros_like(sums_ref)
        cnts_ref[...] = jnp.zeros_like(cnts_ref)

    bv = b_ref[...]
    gids = lax.broadcasted_iota(jnp.int32, (NODE_BLK, 512), 1).astype(jnp.float32)
    oh = jnp.where(gids == bv, 1.0, 0.0)
    hv = h_ref[...]
    dn = (((0,), (0,)), ((), ()))
    sums_ref[...] += lax.dot_general(oh, hv, dn,
                                     preferred_element_type=jnp.float32)
    cnts_ref[...] += lax.dot_general(oh, jnp.ones_like(hv), dn,
                                     preferred_element_type=jnp.float32)

    @pl.when(i == nblk - 1)
    def _():
        g = sums_ref[...] / jnp.maximum(cnts_ref[...], 1.0)
        g = jax.nn.gelu(jnp.dot(g, fw0_ref[...],
                                preferred_element_type=jnp.float32)
                        + fb0_ref[...][0:1])
        g = jax.nn.gelu(jnp.dot(g, fw1_ref[...],
                                preferred_element_type=jnp.float32)
                        + fb1_ref[...][0:1])
        out_ref[...] = jnp.dot(g, ow_ref[...],
                               preferred_element_type=jnp.float32) + ob_ref[...][0:1]


def _tc_readout(h, batch_f, fw0, fb0, fw1, fb1, ow_p, ob_p):
    full = lambda i: (0, 0)
    return pl.pallas_call(
        _readout_body,
        grid=(N // NODE_BLK,),
        in_specs=[
            pl.BlockSpec((NODE_BLK, 128), lambda i: (i, 0)),
            pl.BlockSpec((NODE_BLK, 1), lambda i: (i, 0)),
            pl.BlockSpec((128, 128), full),
            pl.BlockSpec((8, 128), full),
            pl.BlockSpec((128, 128), full),
            pl.BlockSpec((8, 128), full),
            pl.BlockSpec((128, 128), full),
            pl.BlockSpec((8, 128), full),
        ],
        out_specs=pl.BlockSpec((512, 128), full),
        out_shape=jax.ShapeDtypeStruct((512, 128), jnp.float32),
        scratch_shapes=[
            pltpu.VMEM((512, 128), jnp.float32),
            pltpu.VMEM((512, 128), jnp.float32),
        ],
    )(h, batch_f, fw0, fb0, fw1, fb1, ow_p, ob_p)


def _rep8(b):
    return jnp.broadcast_to(b[None, :], (8, b.shape[0])).astype(jnp.float32)


_MESH = plsc.VectorSubcoreMesh(core_axis_name="c", subcore_axis_name="s")
TILES = 32
EPT = E // TILES
NQ = 4
QN = N // NQ
SP_ROWS = QN + 44
STRIPE = SP_ROWS // 16
SLOT = EPT + 128
CC_CHUNK = 5000
CC_VECS = 313
MSG_CHUNK = 48
SUPER = 8


def _sc_pos_gather(pos16, src, dst):
    @functools.partial(
        pl.kernel,
        out_type=[jax.ShapeDtypeStruct((E, 16), jnp.float32),
                  jax.ShapeDtypeStruct((E, 16), jnp.float32)],
        mesh=_MESH,
        scratch_types=[pltpu.VMEM((1000,), jnp.int32),
                       pltpu.VMEM((1000, 16), jnp.float32),
                       pltpu.SemaphoreType.DMA],
        compiler_params=pltpu.CompilerParams(use_tc_tiling_on_sc=False),
    )
    def k(pos_h, src_h, dst_h, ps_o, pd_o, idx_v, rows_v, sem):
        wid = lax.axis_index("c") * 16 + lax.axis_index("s")
        base = wid * EPT
        for idx_h, out_h in ((src_h, ps_o), (dst_h, pd_o)):
            def body(i, _, idx_h=idx_h, out_h=out_h):
                off = base + i * 1000
                pltpu.sync_copy(idx_h.at[pl.ds(off, 1000)], idx_v)
                cps = []
                for kk in range(7):
                    cps.append(pltpu.async_copy(
                        pos_h.at[idx_v.at[pl.ds(kk * 128, 128)]],
                        rows_v.at[pl.ds(kk * 128, 128)], sem))
                cps.append(pltpu.async_copy(
                    pos_h.at[idx_v.at[pl.ds(896, 104)]],
                    rows_v.at[pl.ds(896, 104)], sem))
                for cp in cps:
                    cp.wait()
                pltpu.sync_copy(rows_v, out_h.at[pl.ds(off, 1000)])
                return 0
            lax.fori_loop(0, EPT // 1000, body, 0)

    return k(pos16, src, dst)


def _sc_compact(src, dst, u):
    @functools.partial(
        pl.kernel,
        out_type=[jax.ShapeDtypeStruct((NQ * TILES * SLOT,), jnp.int32),
                  jax.ShapeDtypeStruct((NQ * TILES * SLOT,), jnp.int32),
                  jax.ShapeDtypeStruct((NQ * TILES * SLOT,), jnp.float32),
                  jax.ShapeDtypeStruct((NQ * TILES * 8 + 8,), jnp.int32)],
        mesh=_MESH,
        scratch_types=[pltpu.VMEM((5008,), jnp.int32),
                       pltpu.VMEM((5008,), jnp.int32),
                       pltpu.VMEM((5008,), jnp.float32),
                       pltpu.VMEM((SLOT + 16,), jnp.int32),
                       pltpu.VMEM((SLOT + 16,), jnp.int32),
                       pltpu.VMEM((SLOT + 16,), jnp.float32),
                       pltpu.VMEM((16,), jnp.int32)],
        compiler_params=pltpu.CompilerParams(use_tc_tiling_on_sc=False,
                                             needs_layout_passes=False),
    )
    def k(src_h, dst_h, u_h, csrc_o, cdst_o, cu_o, cnt_o,
          s_in, d_in, u_in, bsrc, bdst, bu, cnt_v):
        wid = lax.axis_index("c") * 16 + lax.axis_index("s")
        base = wid * EPT
        lane = lax.broadcasted_iota(jnp.int32, (16,), 0)
        for q in range(NQ):
            lo = q * QN
            hi = lo + QN

            def chunk_body(c, off, lo=lo, hi=hi):
                pltpu.sync_copy(src_h.at[pl.ds(base + c * CC_CHUNK, CC_CHUNK)],
                                s_in.at[pl.ds(0, CC_CHUNK)])
                pltpu.sync_copy(dst_h.at[pl.ds(base + c * CC_CHUNK, CC_CHUNK)],
                                d_in.at[pl.ds(0, CC_CHUNK)])
                pltpu.sync_copy(u_h.at[pl.ds(base + c * CC_CHUNK, CC_CHUNK)],
                                u_in.at[pl.ds(0, CC_CHUNK)])

                def vec_body(kk, off2):
                    sv = s_in[pl.ds(kk * 16, 16)]
                    dv = d_in[pl.ds(kk * 16, 16)]
                    uv = u_in[pl.ds(kk * 16, 16)]
                    valid = lane < (CC_CHUNK - kk * 16)
                    msk = valid & (dv >= lo) & (dv < hi)
                    mi = msk.astype(jnp.int32)
                    ics = plsc.cumsum(mi)
                    idx = jnp.where(msk, off2 + ics - mi, SLOT + lane)
                    plsc.store_scatter(bsrc, [idx], sv)
                    plsc.store_scatter(bdst, [idx], dv - lo)
                    plsc.store_scatter(bu, [idx], uv)
                    return off2 + ics[15]

                return lax.fori_loop(0, CC_VECS, vec_body, off)

            off = lax.fori_loop(0, EPT // CC_CHUNK, chunk_body, 0)
            dump_d = QN + (lane & 7)
            zero16 = jnp.zeros((16,), jnp.int32)
            ktop16 = jnp.full((16,), float(TAB_K), jnp.float32)
            for j in range(3):
                bsrc[pl.ds(off + j * 16, 16)] = zero16
                bdst[pl.ds(off + j * 16, 16)] = dump_d
                bu[pl.ds(off + j * 16, 16)] = ktop16
            off_pad = ((off + MSG_CHUNK - 1) // MSG_CHUNK) * MSG_CHUNK
            cnt_v[...] = jnp.full((16,), off_pad, jnp.int32)
            pltpu.sync_copy(cnt_v.at[pl.ds(0, 8)],
                            cnt_o.at[pl.ds((q * TILES) * 8 + wid * 8, 8)])
            qbase = q * TILES * SLOT
            pltpu.sync_copy(bsrc.at[pl.ds(0, SLOT)],
                            csrc_o.at[pl.ds(qbase + wid * SLOT, SLOT)])
            pltpu.sync_copy(bdst.at[pl.ds(0, SLOT)],
                            cdst_o.at[pl.ds(qbase + wid * SLOT, SLOT)])
            pltpu.sync_copy(bu.at[pl.ds(0, SLOT)],
                            cu_o.at[pl.ds(qbase + wid * SLOT, SLOT)])

    return k(src, dst, u)


def _sc_message(m, tpair, csrc, cdst, cu, counts, zeros_buf):
    @functools.partial(
        pl.kernel,
        out_type=jax.ShapeDtypeStruct((NQ * SP_ROWS, 128), jnp.float32),
        mesh=_MESH,
        scratch_types=[pltpu.VMEM_SHARED((SP_ROWS, 128), jnp.float32),
                       pltpu.VMEM((NQ * TILES * 8 + 8,), jnp.int32),
                       pltpu.VMEM((SUPER * MSG_CHUNK,), jnp.int32),
                       pltpu.VMEM((SUPER * MSG_CHUNK,), jnp.int32),
                       pltpu.VMEM((SUPER * MSG_CHUNK,), jnp.float32),
                       pltpu.VMEM((SUPER * MSG_CHUNK,), jnp.int32),
                       pltpu.VMEM((MSG_CHUNK, 128), jnp.float32),
                       pltpu.VMEM((MSG_CHUNK, 128), jnp.float32),
                       pltpu.VMEM((MSG_CHUNK, 128), jnp.float32),
                       pltpu.VMEM((MSG_CHUNK, 128), jnp.float32),
                       pltpu.SemaphoreType.DMA,
                       pltpu.SemaphoreType.DMA],
        compiler_params=pltpu.CompilerParams(needs_layout_passes=False),
    )
    def k(m_h, tp_h, csrc_h, cdst_h, cu_h, cnt_h, zeros_h, agg_h,
          sharedA, cnt_v, src_v, dst_v, u_v, k_v, mrow0, prow0, mrow1, prow1,
          semA, semB):
        cid = lax.axis_index("c")
        sid = lax.axis_index("s")
        lane = lax.broadcasted_iota(jnp.int32, (16,), 0)
        rep_off = (lane & (TAB_REPS - 1)) * TAB_ROWS
        pltpu.sync_copy(cnt_h, cnt_v)
        mrows = (mrow0, mrow1)
        prows = (prow0, prow1)
        sems = (semA, semB)

        def mul_scatter(slot, sub):
            mrow, prow = mrows[slot], prows[slot]

            def mul(j, _):
                for cc in range(8):
                    sl = pl.ds(cc * 16, 16)
                    mrow[j, sl] = mrow[j, sl] * prow[j, sl]
                return 0

            lax.fori_loop(0, MSG_CHUNK, mul, 0)
            pltpu.sync_copy(mrow,
                            sharedA.at[dst_v.at[pl.ds(sub * MSG_CHUNK,
                                                      MSG_CHUNK)]],
                            add=True)

        def issue(sub, slot):
            s = pl.ds(sub * MSG_CHUNK, MSG_CHUNK)
            c1 = pltpu.async_copy(m_h.at[src_v.at[s]], mrows[slot],
                                  sems[slot])
            c2 = pltpu.async_copy(tp_h.at[k_v.at[s]], prows[slot],
                                  sems[slot])
            return (c1, c2)

        for qj in range(NQ // 2):
            q = cid * (NQ // 2) + qj
            pltpu.sync_copy(zeros_h,
                            sharedA.at[pl.ds(sid * STRIPE, STRIPE)])
            plsc.subcore_barrier()
            for tj in range(2):
                t = tj * 16 + sid
                nq = cnt_v[pl.ds((q * TILES + t) * 8, 16)][0]
                trips = nq // MSG_CHUNK
                nss = trips // SUPER

                def load_idx(b, count):
                    pltpu.sync_copy(csrc_h.at[pl.ds(b, count)],
                                    src_v.at[pl.ds(0, count)])
                    pltpu.sync_copy(cdst_h.at[pl.ds(b, count)],
                                    dst_v.at[pl.ds(0, count)])
                    pltpu.sync_copy(cu_h.at[pl.ds(b, count)],
                                    u_v.at[pl.ds(0, count)])
                    for g in range(count // 16):
                        u16 = u_v[pl.ds(g * 16, 16)]
                        k16 = (u16 + 0.5).astype(jnp.int32) + rep_off
                        k_v[pl.ds(g * 16, 16)] = k16

                def ss_body(si, _, t=t, q=q):
                    b = q * TILES * SLOT + t * SLOT + si * (SUPER * MSG_CHUNK)
                    load_idx(b, SUPER * MSG_CHUNK)
                    cps = {0: issue(0, 0)}
                    for sub in range(SUPER):
                        slot = sub & 1
                        if sub + 1 < SUPER:
                            cps[sub + 1] = issue(sub + 1, slot ^ 1)
                        cps[sub][0].wait()
                        cps[sub][1].wait()
                        mul_scatter(slot, sub)
                    return 0

                lax.fori_loop(0, nss, ss_body, 0)

                def tail_body(ci, _, t=t, q=q):
                    b = (q * TILES * SLOT + t * SLOT + ci * MSG_CHUNK)
                    load_idx(b, MSG_CHUNK)
                    c1, c2 = issue(0, 0)
                    c1.wait()
                    c2.wait()
                    mul_scatter(0, 0)
                    return 0

                lax.fori_loop(nss * SUPER, trips, tail_body, 0)
            plsc.subcore_barrier()
            row0 = q * SP_ROWS + sid * STRIPE
            pltpu.sync_copy(sharedA.at[pl.ds(sid * STRIPE, STRIPE)],
                            agg_h.at[pl.ds(row0, STRIPE)])
            plsc.subcore_barrier()

    return k(m, tpair, csrc, cdst, cu, counts, zeros_buf)


def kernel(pos, emb, Wf1, bf1, Wf2, bf2, W1, W2, b2, W3, b3, fcW, fcb,
           outW, outb, x, edge_index, batch):
    src = edge_index[0]
    dst = edge_index[1]
    x_f = x.astype(jnp.float32)
    batch_f = batch.astype(jnp.float32)[:, None]
    emb_p = jnp.pad(emb, ((0, 128 - NTYPES), (0, 0)))
    wf1_p = jnp.pad(Wf1, ((0, 0), (0, 128 - G), (0, 0)))
    ow_p = jnp.pad(outW, ((0, 0), (0, 128 - NCLS)))
    ob_p = _rep8(jnp.pad(outb, (0, 128 - NCLS)))

    h, m = _tc_embed(x_f, emb_p, W1[0])

    pos16 = jnp.pad(pos, ((0, 0), (0, 13)))
    pos_s, pos_d = _sc_pos_gather(pos16, src, dst)
    seg = ((jnp.arange(128)[:, None] // 16 == jnp.arange(128)[None, :])
           & (jnp.arange(128)[None, :] < 8)).astype(jnp.float32)
    ps8 = jnp.reshape(pos_s, (UROWS, 128))
    pd8 = jnp.reshape(pos_d, (UROWS, 128))
    u8 = _tc_edge_u(ps8, pd8, seg)
    u1 = jnp.reshape(u8, (E,))
    csrc, cdst, cu, counts = _sc_compact(src, dst, u1)
    zeros_buf = jnp.zeros((STRIPE, 128), jnp.float32)

    dgrid = jnp.broadcast_to(
        (jnp.arange(TAB_BUILD, dtype=jnp.float32) * TAB_STEP)[:, None],
        (TAB_BUILD, 128))

    for l in range(L):
        tab = _tc_table(dgrid, wf1_p[l], _rep8(bf1[l]), Wf2[l], _rep8(bf2[l]))
        trep = jnp.tile(tab[0:TAB_ROWS], (TAB_REPS, 1))
        agg_full = _sc_message(m, trep, csrc, cdst, cu, counts, zeros_buf)
        agg = jnp.concatenate(
            [agg_full[q * SP_ROWS:q * SP_ROWS + QN] for q in range(NQ)], axis=0)
        w1n = W1[l + 1] if l + 1 < L else None
        if w1n is not None:
            h, m = _tc_node_update(agg, h, W2[l], _rep8(b2[l]), W3[l],
                                   _rep8(b3[l]), w1n)
        else:
            h = _tc_node_update(agg, h, W2[l], _rep8(b2[l]), W3[l],
                                _rep8(b3[l]), None)

    out = _tc_readout(h, batch_f, fcW[0], _rep8(fcb[0]), fcW[1],
                      _rep8(fcb[1]), ow_p, ob_p)
    return out[:NG, :NCLS]


# speedup vs baseline: 1.1133x; 1.0734x over previous
import functools
import math

import jax
import jax.numpy as jnp
from jax import lax
from jax.experimental import pallas as pl
from jax.experimental.pallas import tpu as pltpu
from jax.experimental.pallas import tpu_sc as plsc

N = 50000
E = 800000
L = 3
H = 128
F = 128
G = 50
NG = 500
NTYPES = 100
CUTOFF = 5.0
NFC = 2
NCLS = 1

LN2 = math.log(2.0)

NODE_BLK = 5000
EDGE_BLK = 10000

TAB_K = 16384
TAB_STEP = CUTOFF / TAB_K
TAB_ROWS = 16512
TAB_BUILD = TAB_ROWS + 8
TAB_REPS = 8


def _ssp(v):
    return jax.nn.softplus(v) - LN2


def _embed_body(x_ref, emb_ref, w1_ref, h_ref, m_ref):
    xv = x_ref[...]
    ids = lax.broadcasted_iota(jnp.int32, (NODE_BLK, 128), 1).astype(jnp.float32)
    oh = jnp.where(ids == xv, 1.0, 0.0)
    h = jnp.dot(oh, emb_ref[...], preferred_element_type=jnp.float32)
    h_ref[...] = h
    m_ref[...] = jnp.dot(h, w1_ref[...], preferred_element_type=jnp.float32)


def _tc_embed(x_f, emb_p, w1_0):
    return pl.pallas_call(
        _embed_body,
        grid=(N // NODE_BLK,),
        in_specs=[
            pl.BlockSpec((NODE_BLK, 1), lambda i: (i, 0)),
            pl.BlockSpec((128, 128), lambda i: (0, 0)),
            pl.BlockSpec((128, 128), lambda i: (0, 0)),
        ],
        out_specs=[
            pl.BlockSpec((NODE_BLK, 128), lambda i: (i, 0)),
            pl.BlockSpec((NODE_BLK, 128), lambda i: (i, 0)),
        ],
        out_shape=[
            jax.ShapeDtypeStruct((N, 128), jnp.float32),
            jax.ShapeDtypeStruct((N, 128), jnp.float32),
        ],
    )(x_f, emb_p, w1_0)


def _table_body(d_ref, wf1_ref, bf1_ref, wf2_ref, bf2_ref, out_ref):
    d = d_ref[...]
    step = CUTOFF / (G - 1)
    offs = lax.broadcasted_iota(jnp.int32, (1, 128), 1).astype(jnp.float32) * step
    coeff = -0.5 / (step * step)
    rbf = jnp.exp(coeff * (d - offs) ** 2)
    t = _ssp(jnp.dot(rbf, wf1_ref[...], preferred_element_type=jnp.float32)
             + bf1_ref[...][0:1])
    w = (jnp.dot(t, wf2_ref[...], preferred_element_type=jnp.float32)
         + bf2_ref[...][0:1])
    c = 0.5 * (jnp.cos(d * (math.pi / CUTOFF)) + 1.0)
    c = jnp.where(d < CUTOFF, c, 0.0)
    out_ref[...] = w * c


def _tc_table(dgrid, wf1_l, bf1_l, wf2_l, bf2_l):
    return pl.pallas_call(
        _table_body,
        grid=(1,),
        in_specs=[
            pl.BlockSpec((TAB_BUILD, 128), lambda i: (0, 0)),
            pl.BlockSpec((128, 128), lambda i: (0, 0)),
            pl.BlockSpec((8, 128), lambda i: (0, 0)),
            pl.BlockSpec((128, 128), lambda i: (0, 0)),
            pl.BlockSpec((8, 128), lambda i: (0, 0)),
        ],
        out_specs=pl.BlockSpec((TAB_BUILD, 128), lambda i: (0, 0)),
        out_shape=jax.ShapeDtypeStruct((TAB_BUILD, 128), jnp.float32),
    )(dgrid, wf1_l, bf1_l, wf2_l, bf2_l)


UROWS = E // 8


def _edge_u_body(ps_ref, pd_ref, seg_ref, out_ref):
    diff = ps_ref[...] - pd_ref[...]
    d2 = jnp.dot(diff * diff, seg_ref[...],
                 preferred_element_type=jnp.float32) + 1e-12
    u = jnp.minimum(jnp.sqrt(d2) * (1.0 / TAB_STEP), float(TAB_K))
    out_ref[...] = u[:, 0:8]


def _tc_edge_u(ps8, pd8, seg):
    blk = 10000
    return pl.pallas_call(
        _edge_u_body,
        grid=(UROWS // blk,),
        in_specs=[
            pl.BlockSpec((blk, 128), lambda i: (i, 0)),
            pl.BlockSpec((blk, 128), lambda i: (i, 0)),
            pl.BlockSpec((128, 128), lambda i: (0, 0)),
        ],
        out_specs=pl.BlockSpec((blk, 8), lambda i: (i, 0)),
        out_shape=jax.ShapeDtypeStruct((UROWS, 8), jnp.float32),
    )(ps8, pd8, seg)


def _node_body_mid(agg_ref, h_ref, w2_ref, b2_ref, w3_ref, b3_ref, w1n_ref,
                   hn_ref, mn_ref):
    v = _ssp(jnp.dot(agg_ref[...], w2_ref[...],
                     preferred_element_type=jnp.float32) + b2_ref[...][0:1])
    hn = h_ref[...] + jnp.dot(v, w3_ref[...],
                              preferred_element_type=jnp.float32) + b3_ref[...][0:1]
    hn_ref[...] = hn
    mn_ref[...] = jnp.dot(hn, w1n_ref[...], preferred_element_type=jnp.float32)


def _node_body_last(agg_ref, h_ref, w2_ref, b2_ref, w3_ref, b3_ref, hn_ref):
    v = _ssp(jnp.dot(agg_ref[...], w2_ref[...],
                     preferred_element_type=jnp.float32) + b2_ref[...][0:1])
    hn_ref[...] = h_ref[...] + jnp.dot(v, w3_ref[...],
                                       preferred_element_type=jnp.float32) + b3_ref[...][0:1]


def _tc_node_update(agg, h, w2_l, b2_l, w3_l, b3_l, w1_next):
    full = lambda i: (0, 0)
    blk = lambda i: (i, 0)
    if w1_next is not None:
        return pl.pallas_call(
            _node_body_mid,
            grid=(N // NODE_BLK,),
            in_specs=[
                pl.BlockSpec((NODE_BLK, 128), blk),
                pl.BlockSpec((NODE_BLK, 128), blk),
                pl.BlockSpec((128, 128), full),
                pl.BlockSpec((8, 128), full),
                pl.BlockSpec((128, 128), full),
                pl.BlockSpec((8, 128), full),
                pl.BlockSpec((128, 128), full),
            ],
            out_specs=[
                pl.BlockSpec((NODE_BLK, 128), blk),
                pl.BlockSpec((NODE_BLK, 128), blk),
            ],
            out_shape=[
                jax.ShapeDtypeStruct((N, 128), jnp.float32),
                jax.ShapeDtypeStruct((N, 128), jnp.float32),
            ],
        )(agg, h, w2_l, b2_l, w3_l, b3_l, w1_next)
    return pl.pallas_call(
        _node_body_last,
        grid=(N // NODE_BLK,),
        in_specs=[
            pl.BlockSpec((NODE_BLK, 128), blk),
            pl.BlockSpec((NODE_BLK, 128), blk),
            pl.BlockSpec((128, 128), full),
            pl.BlockSpec((8, 128), full),
            pl.BlockSpec((128, 128), full),
            pl.BlockSpec((8, 128), full),
        ],
        out_specs=pl.BlockSpec((NODE_BLK, 128), blk),
        out_shape=jax.ShapeDtypeStruct((N, 128), jnp.float32),
    )(agg, h, w2_l, b2_l, w3_l, b3_l)


def _readout_body(h_ref, b_ref, fw0_ref, fb0_ref, fw1_ref, fb1_ref,
                  ow_ref, ob_ref, out_ref, sums_ref, cnts_ref):
    i = pl.program_id(0)
    nblk = pl.num_programs(0)

    @pl.when(i == 0)
    def _():
        sums_ref[...] = jnp.zeros_like(sums_ref)
        cnts_ref[...] = jnp.zeros_like(cnts_ref)

    bv = b_ref[...]
    gids = lax.broadcasted_iota(jnp.int32, (NODE_BLK, 512), 1).astype(jnp.float32)
    oh = jnp.where(gids == bv, 1.0, 0.0)
    hv = h_ref[...]
    dn = (((0,), (0,)), ((), ()))
    sums_ref[...] += lax.dot_general(oh, hv, dn,
                                     preferred_element_type=jnp.float32)
    cnts_ref[...] += lax.dot_general(oh, jnp.ones_like(hv), dn,
                                     preferred_element_type=jnp.float32)

    @pl.when(i == nblk - 1)
    def _():
        g = sums_ref[...] / jnp.maximum(cnts_ref[...], 1.0)
        g = jax.nn.gelu(jnp.dot(g, fw0_ref[...],
                                preferred_element_type=jnp.float32)
                        + fb0_ref[...][0:1])
        g = jax.nn.gelu(jnp.dot(g, fw1_ref[...],
                                preferred_element_type=jnp.float32)
                        + fb1_ref[...][0:1])
        out_ref[...] = jnp.dot(g, ow_ref[...],
                               preferred_element_type=jnp.float32) + ob_ref[...][0:1]


def _tc_readout(h, batch_f, fw0, fb0, fw1, fb1, ow_p, ob_p):
    full = lambda i: (0, 0)
    return pl.pallas_call(
        _readout_body,
        grid=(N // NODE_BLK,),
        in_specs=[
            pl.BlockSpec((NODE_BLK, 128), lambda i: (i, 0)),
            pl.BlockSpec((NODE_BLK, 1), lambda i: (i, 0)),
            pl.BlockSpec((128, 128), full),
            pl.BlockSpec((8, 128), full),
            pl.BlockSpec((128, 128), full),
            pl.BlockSpec((8, 128), full),
            pl.BlockSpec((128, 128), full),
            pl.BlockSpec((8, 128), full),
        ],
        out_specs=pl.BlockSpec((512, 128), full),
        out_shape=jax.ShapeDtypeStruct((512, 128), jnp.float32),
        scratch_shapes=[
            pltpu.VMEM((512, 128), jnp.float32),
            pltpu.VMEM((512, 128), jnp.float32),
        ],
    )(h, batch_f, fw0, fb0, fw1, fb1, ow_p, ob_p)


def _rep8(b):
    return jnp.broadcast_to(b[None, :], (8, b.shape[0])).astype(jnp.float32)


_MESH = plsc.VectorSubcoreMesh(core_axis_name="c", subcore_axis_name="s")
TILES = 32
EPT = E // TILES
NQ = 4
QN = N // NQ
SP_ROWS = QN + 44
STRIPE = SP_ROWS // 16
SLOT = EPT + 128
CC_CHUNK = 5000
CC_VECS = 313
MSG_CHUNK = 48
SUPER = 16


def _sc_pos_gather(pos16, src, dst):
    @functools.partial(
        pl.kernel,
        out_type=[jax.ShapeDtypeStruct((E, 16), jnp.float32),
                  jax.ShapeDtypeStruct((E, 16), jnp.float32)],
        mesh=_MESH,
        scratch_types=[pltpu.VMEM((1000,), jnp.int32),
                       pltpu.VMEM((1000, 16), jnp.float32),
                       pltpu.SemaphoreType.DMA],
        compiler_params=pltpu.CompilerParams(use_tc_tiling_on_sc=False),
    )
    def k(pos_h, src_h, dst_h, ps_o, pd_o, idx_v, rows_v, sem):
        wid = lax.axis_index("c") * 16 + lax.axis_index("s")
        base = wid * EPT
        for idx_h, out_h in ((src_h, ps_o), (dst_h, pd_o)):
            def body(i, _, idx_h=idx_h, out_h=out_h):
                off = base + i * 1000
                pltpu.sync_copy(idx_h.at[pl.ds(off, 1000)], idx_v)
                cps = []
                for kk in range(7):
                    cps.append(pltpu.async_copy(
                        pos_h.at[idx_v.at[pl.ds(kk * 128, 128)]],
                        rows_v.at[pl.ds(kk * 128, 128)], sem))
                cps.append(pltpu.async_copy(
                    pos_h.at[idx_v.at[pl.ds(896, 104)]],
                    rows_v.at[pl.ds(896, 104)], sem))
                for cp in cps:
                    cp.wait()
                pltpu.sync_copy(rows_v, out_h.at[pl.ds(off, 1000)])
                return 0
            lax.fori_loop(0, EPT // 1000, body, 0)

    return k(pos16, src, dst)


def _sc_compact(src, dst, u):
    @functools.partial(
        pl.kernel,
        out_type=[jax.ShapeDtypeStruct((NQ * TILES * SLOT,), jnp.int32),
                  jax.ShapeDtypeStruct((NQ * TILES * SLOT,), jnp.int32),
                  jax.ShapeDtypeStruct((NQ * TILES * SLOT,), jnp.float32),
                  jax.ShapeDtypeStruct((NQ * TILES * 8 + 8,), jnp.int32)],
        mesh=_MESH,
        scratch_types=[pltpu.VMEM((5008,), jnp.int32),
                       pltpu.VMEM((5008,), jnp.int32),
                       pltpu.VMEM((5008,), jnp.float32),
                       pltpu.VMEM((SLOT + 16,), jnp.int32),
                       pltpu.VMEM((SLOT + 16,), jnp.int32),
                       pltpu.VMEM((SLOT + 16,), jnp.float32),
                       pltpu.VMEM((16,), jnp.int32)],
        compiler_params=pltpu.CompilerParams(use_tc_tiling_on_sc=False,
                                             needs_layout_passes=False),
    )
    def k(src_h, dst_h, u_h, csrc_o, cdst_o, cu_o, cnt_o,
          s_in, d_in, u_in, bsrc, bdst, bu, cnt_v):
        wid = lax.axis_index("c") * 16 + lax.axis_index("s")
        base = wid * EPT
        lane = lax.broadcasted_iota(jnp.int32, (16,), 0)
        for q in range(NQ):
            lo = q * QN
            hi = lo + QN

            def chunk_body(c, off, lo=lo, hi=hi):
                pltpu.sync_copy(src_h.at[pl.ds(base + c * CC_CHUNK, CC_CHUNK)],
                                s_in.at[pl.ds(0, CC_CHUNK)])
                pltpu.sync_copy(dst_h.at[pl.ds(base + c * CC_CHUNK, CC_CHUNK)],
                                d_in.at[pl.ds(0, CC_CHUNK)])
                pltpu.sync_copy(u_h.at[pl.ds(base + c * CC_CHUNK, CC_CHUNK)],
                                u_in.at[pl.ds(0, CC_CHUNK)])

                def vec_body(kk, off2):
                    sv = s_in[pl.ds(kk * 16, 16)]
                    dv = d_in[pl.ds(kk * 16, 16)]
                    uv = u_in[pl.ds(kk * 16, 16)]
                    valid = lane < (CC_CHUNK - kk * 16)
                    msk = valid & (dv >= lo) & (dv < hi)
                    mi = msk.astype(jnp.int32)
                    ics = plsc.cumsum(mi)
                    idx = jnp.where(msk, off2 + ics - mi, SLOT + lane)
                    plsc.store_scatter(bsrc, [idx], sv)
                    plsc.store_scatter(bdst, [idx], dv - lo)
                    plsc.store_scatter(bu, [idx], uv)
                    return off2 + ics[15]

                return lax.fori_loop(0, CC_VECS, vec_body, off)

            off = lax.fori_loop(0, EPT // CC_CHUNK, chunk_body, 0)
            dump_d = QN + (lane & 7)
            zero16 = jnp.zeros((16,), jnp.int32)
            ktop16 = jnp.full((16,), float(TAB_K), jnp.float32)
            for j in range(3):
                bsrc[pl.ds(off + j * 16, 16)] = zero16
                bdst[pl.ds(off + j * 16, 16)] = dump_d
                bu[pl.ds(off + j * 16, 16)] = ktop16
            off_pad = ((off + MSG_CHUNK - 1) // MSG_CHUNK) * MSG_CHUNK
            cnt_v[...] = jnp.full((16,), off_pad, jnp.int32)
            pltpu.sync_copy(cnt_v.at[pl.ds(0, 8)],
                            cnt_o.at[pl.ds((q * TILES) * 8 + wid * 8, 8)])
            qbase = q * TILES * SLOT
            pltpu.sync_copy(bsrc.at[pl.ds(0, SLOT)],
                            csrc_o.at[pl.ds(qbase + wid * SLOT, SLOT)])
            pltpu.sync_copy(bdst.at[pl.ds(0, SLOT)],
                            cdst_o.at[pl.ds(qbase + wid * SLOT, SLOT)])
            pltpu.sync_copy(bu.at[pl.ds(0, SLOT)],
                            cu_o.at[pl.ds(qbase + wid * SLOT, SLOT)])

    return k(src, dst, u)


def _sc_message(m, tpair, csrc, cdst, cu, counts, zeros_buf):
    @functools.partial(
        pl.kernel,
        out_type=jax.ShapeDtypeStruct((NQ * SP_ROWS, 128), jnp.float32),
        mesh=_MESH,
        scratch_types=[pltpu.VMEM_SHARED((SP_ROWS, 128), jnp.float32),
                       pltpu.VMEM((NQ * TILES * 8 + 8,), jnp.int32),
                       pltpu.VMEM((SUPER * MSG_CHUNK,), jnp.int32),
                       pltpu.VMEM((SUPER * MSG_CHUNK,), jnp.int32),
                       pltpu.VMEM((SUPER * MSG_CHUNK,), jnp.float32),
                       pltpu.VMEM((SUPER * MSG_CHUNK,), jnp.int32),
                       pltpu.VMEM((MSG_CHUNK, 128), jnp.float32),
                       pltpu.VMEM((MSG_CHUNK, 128), jnp.float32),
                       pltpu.VMEM((MSG_CHUNK, 128), jnp.float32),
                       pltpu.VMEM((MSG_CHUNK, 128), jnp.float32),
                       pltpu.SemaphoreType.DMA,
                       pltpu.SemaphoreType.DMA,
                       pltpu.SemaphoreType.DMA,
                       pltpu.SemaphoreType.DMA],
        compiler_params=pltpu.CompilerParams(needs_layout_passes=False),
    )
    def k(m_h, tp_h, csrc_h, cdst_h, cu_h, cnt_h, zeros_h, agg_h,
          sharedA, cnt_v, src_v, dst_v, u_v, k_v, mrow0, prow0, mrow1, prow1,
          semA, semB, semS0, semS1):
        cid = lax.axis_index("c")
        sid = lax.axis_index("s")
        lane = lax.broadcasted_iota(jnp.int32, (16,), 0)
        rep_off = (lane & (TAB_REPS - 1)) * TAB_ROWS
        pltpu.sync_copy(cnt_h, cnt_v)
        mrows = (mrow0, mrow1)
        prows = (prow0, prow1)
        sems = (semA, semB)
        ssems = (semS0, semS1)

        def mul_scatter(slot, sub):
            mrow, prow = mrows[slot], prows[slot]

            def mul(j, _):
                for cc in range(8):
                    sl = pl.ds(cc * 16, 16)
                    mrow[j, sl] = mrow[j, sl] * prow[j, sl]
                return 0

            lax.fori_loop(0, MSG_CHUNK, mul, 0)
            return pltpu.async_copy(
                mrow,
                sharedA.at[dst_v.at[pl.ds(sub * MSG_CHUNK, MSG_CHUNK)]],
                ssems[slot], add=True)

        def issue(sub, slot):
            s = pl.ds(sub * MSG_CHUNK, MSG_CHUNK)
            c1 = pltpu.async_copy(m_h.at[src_v.at[s]], mrows[slot],
                                  sems[slot])
            c2 = pltpu.async_copy(tp_h.at[k_v.at[s]], prows[slot],
                                  sems[slot])
            return (c1, c2)

        for qj in range(NQ // 2):
            q = cid * (NQ // 2) + qj
            pltpu.sync_copy(zeros_h,
                            sharedA.at[pl.ds(sid * STRIPE, STRIPE)])
            plsc.subcore_barrier()
            for tj in range(2):
                t = tj * 16 + sid
                nq = cnt_v[pl.ds((q * TILES + t) * 8, 16)][0]
                trips = nq // MSG_CHUNK
                nss = trips // SUPER

                def load_idx(b, count):
                    pltpu.sync_copy(csrc_h.at[pl.ds(b, count)],
                                    src_v.at[pl.ds(0, count)])
                    pltpu.sync_copy(cdst_h.at[pl.ds(b, count)],
                                    dst_v.at[pl.ds(0, count)])
                    pltpu.sync_copy(cu_h.at[pl.ds(b, count)],
                                    u_v.at[pl.ds(0, count)])
                    for g in range(count // 16):
                        u16 = u_v[pl.ds(g * 16, 16)]
                        k16 = (u16 + 0.5).astype(jnp.int32) + rep_off
                        k_v[pl.ds(g * 16, 16)] = k16

                def ss_body(si, _, t=t, q=q):
                    b = q * TILES * SLOT + t * SLOT + si * (SUPER * MSG_CHUNK)
                    load_idx(b, SUPER * MSG_CHUNK)
                    cps = {0: issue(0, 0)}
                    scats = {}
                    for sub in range(SUPER):
                        slot = sub & 1
                        if sub + 1 < SUPER:
                            if sub - 1 in scats:
                                scats[sub - 1].wait()
                            cps[sub + 1] = issue(sub + 1, slot ^ 1)
                        cps[sub][0].wait()
                        cps[sub][1].wait()
                        scats[sub] = mul_scatter(slot, sub)
                    scats[SUPER - 2].wait()
                    scats[SUPER - 1].wait()
                    return 0

                lax.fori_loop(0, nss, ss_body, 0)

                def tail_body(ci, _, t=t, q=q):
                    b = (q * TILES * SLOT + t * SLOT + ci * MSG_CHUNK)
                    load_idx(b, MSG_CHUNK)
                    c1, c2 = issue(0, 0)
                    c1.wait()
                    c2.wait()
                    mul_scatter(0, 0).wait()
                    return 0

                lax.fori_loop(nss * SUPER, trips, tail_body, 0)
            plsc.subcore_barrier()
            row0 = q * SP_ROWS + sid * STRIPE
            pltpu.sync_copy(sharedA.at[pl.ds(sid * STRIPE, STRIPE)],
                            agg_h.at[pl.ds(row0, STRIPE)])
            plsc.subcore_barrier()

    return k(m, tpair, csrc, cdst, cu, counts, zeros_buf)


def kernel(pos, emb, Wf1, bf1, Wf2, bf2, W1, W2, b2, W3, b3, fcW, fcb,
           outW, outb, x, edge_index, batch):
    src = edge_index[0]
    dst = edge_index[1]
    x_f = x.astype(jnp.float32)
    batch_f = batch.astype(jnp.float32)[:, None]
    emb_p = jnp.pad(emb, ((0, 128 - NTYPES), (0, 0)))
    wf1_p = jnp.pad(Wf1, ((0, 0), (0, 128 - G), (0, 0)))
    ow_p = jnp.pad(outW, ((0, 0), (0, 128 - NCLS)))
    ob_p = _rep8(jnp.pad(outb, (0, 128 - NCLS)))

    h, m = _tc_embed(x_f, emb_p, W1[0])

    pos16 = jnp.pad(pos, ((0, 0), (0, 13)))
    pos_s, pos_d = _sc_pos_gather(pos16, src, dst)
    seg = ((jnp.arange(128)[:, None] // 16 == jnp.arange(128)[None, :])
           & (jnp.arange(128)[None, :] < 8)).astype(jnp.float32)
    ps8 = jnp.reshape(pos_s, (UROWS, 128))
    pd8 = jnp.reshape(pos_d, (UROWS, 128))
    u8 = _tc_edge_u(ps8, pd8, seg)
    u1 = jnp.reshape(u8, (E,))
    csrc, cdst, cu, counts = _sc_compact(src, dst, u1)
    zeros_buf = jnp.zeros((STRIPE, 128), jnp.float32)

    dgrid = jnp.broadcast_to(
        (jnp.arange(TAB_BUILD, dtype=jnp.float32) * TAB_STEP)[:, None],
        (TAB_BUILD, 128))

    for l in range(L):
        tab = _tc_table(dgrid, wf1_p[l], _rep8(bf1[l]), Wf2[l], _rep8(bf2[l]))
        trep = jnp.tile(tab[0:TAB_ROWS], (TAB_REPS, 1))
        agg_full = _sc_message(m, trep, csrc, cdst, cu, counts, zeros_buf)
        agg = jnp.concatenate(
            [agg_full[q * SP_ROWS:q * SP_ROWS + QN] for q in range(NQ)], axis=0)
        w1n = W1[l + 1] if l + 1 < L else None
        if w1n is not None:
            h, m = _tc_node_update(agg, h, W2[l], _rep8(b2[l]), W3[l],
                                   _rep8(b3[l]), w1n)
        else:
            h = _tc_node_update(agg, h, W2[l], _rep8(b2[l]), W3[l],
                                _rep8(b3[l]), None)

    out = _tc_readout(h, batch_f, fcW[0], _rep8(fcb[0]), fcW[1],
                      _rep8(fcb[1]), ow_p, ob_p)
    return out[:NG, :NCLS]
